# Initial kernel scaffold; baseline (speedup 1.0000x reference)
import jax
import jax.numpy as jnp
from jax.experimental import pallas as pl


def kernel(x, embeddings):
    raise NotImplementedError("write your pallas kernel here")


# speedup vs baseline: 1.0935x; 1.0935x over previous
import functools

import jax
import jax.numpy as jnp
from jax import lax
from jax.experimental import pallas as pl
from jax.experimental.pallas import tpu as pltpu, tpu_sc as plsc

VOCAB = 1_000_000
D = 32
L = 128
NC, NS = 2, 16
NW = NC * NS

BATCH, HIST = 16384, 50
B = BATCH * HIST
ROWS = B // L
ROWS_PER_W = ROWS // NW
RPC = 8
CHUNKS = ROWS_PER_W // RPC


def _gather_body(table_hbm, idx_hbm, out_hbm, idx_v, rows_v, gsem):
    wid = lax.axis_index("s") * NC + lax.axis_index("c")
    w_row0 = wid * ROWS_PER_W

    def chunk(g, _):
        row0 = pl.multiple_of(w_row0 + g * RPC, 8)
        pltpu.sync_copy(idx_hbm.at[pl.ds(row0, RPC)], idx_v)
        copies = [
            pltpu.async_copy(
                table_hbm.at[idx_v.at[j]],
                rows_v.at[pl.ds(j * L, L)],
                gsem,
            )
            for j in range(RPC)
        ]
        for c in copies:
            c.wait()
        pltpu.sync_copy(rows_v, out_hbm.at[pl.ds(row0 * L, RPC * L)])
        return ()

    lax.fori_loop(0, CHUNKS, chunk, (), unroll=False)


@functools.partial(jax.jit, static_argnames=())
def kernel(x, embeddings):
    mesh = plsc.VectorSubcoreMesh(core_axis_name="c", subcore_axis_name="s")
    run = pl.kernel(
        _gather_body,
        out_type=jax.ShapeDtypeStruct((B, D), jnp.float32),
        mesh=mesh,
        scratch_types=[
            pltpu.VMEM((RPC, L), jnp.int32),
            pltpu.VMEM((RPC * L, D), jnp.float32),
            pltpu.SemaphoreType.DMA,
        ],
        compiler_params=pltpu.CompilerParams(use_tc_tiling_on_sc=False),
    )
    out = run(embeddings, x.reshape(ROWS, L))
    return out.reshape(BATCH, HIST, D)


# speedup vs baseline: 1.1003x; 1.0062x over previous
import functools

import jax
import jax.numpy as jnp
from jax import lax
from jax.experimental import pallas as pl
from jax.experimental.pallas import tpu as pltpu, tpu_sc as plsc

VOCAB = 1_000_000
D = 32
L = 128
NC, NS = 2, 16
NW = NC * NS

BATCH, HIST = 16384, 50
B = BATCH * HIST
ROWS = B // L
ROWS_PER_W = ROWS // NW
RPC = 8
CHUNKS = ROWS_PER_W // RPC


CL = RPC * L


def _gather_body(table_hbm, idx_hbm, out_hbm, idx_v, rows_v, gsem, osem):
    wid = lax.axis_index("s") * NC + lax.axis_index("c")
    w_row0 = wid * ROWS_PER_W

    def fire(c, buf):
        row0 = pl.multiple_of(w_row0 + c * RPC, 8)
        ib = pl.multiple_of(buf * RPC, 8)
        pltpu.sync_copy(idx_hbm.at[pl.ds(row0, RPC)], idx_v.at[pl.ds(ib, RPC)])
        for j in range(RPC):
            pltpu.async_copy(
                table_hbm.at[idx_v.at[ib + j]],
                rows_v.at[pl.ds(pl.multiple_of(buf * CL + j * L, 8), L)],
                gsem,
            )

    def wait_gathers(buf):
        pltpu.make_async_copy(
            table_hbm.at[pl.ds(0, CL)],
            rows_v.at[pl.ds(pl.multiple_of(buf * CL, 8), CL)],
            gsem,
        ).wait()

    def out_copy(c, buf):
        row0 = pl.multiple_of(w_row0 + c * RPC, 8)
        return pltpu.make_async_copy(
            rows_v.at[pl.ds(pl.multiple_of(buf * CL, 8), CL)],
            out_hbm.at[pl.ds(row0 * L, CL)],
            osem,
        )

    fire(0, 0)

    def body(c, _):
        buf = lax.rem(c, 2)
        wait_gathers(buf)

        @pl.when(c >= 1)
        def _():
            out_copy(c - 1, 1 - buf).wait()

        @pl.when(c + 1 < CHUNKS)
        def _():
            fire(c + 1, 1 - buf)

        out_copy(c, buf).start()
        return ()

    lax.fori_loop(0, CHUNKS, body, (), unroll=False)
    out_copy(CHUNKS - 1, lax.rem(CHUNKS - 1, 2)).wait()


@functools.partial(jax.jit, static_argnames=())
def kernel(x, embeddings):
    mesh = plsc.VectorSubcoreMesh(core_axis_name="c", subcore_axis_name="s")
    run = pl.kernel(
        _gather_body,
        out_type=jax.ShapeDtypeStruct((B, D), jnp.float32),
        mesh=mesh,
        scratch_types=[
            pltpu.VMEM((2 * RPC, L), jnp.int32),
            pltpu.VMEM((2 * CL, D), jnp.float32),
            pltpu.SemaphoreType.DMA,
            pltpu.SemaphoreType.DMA,
        ],
        compiler_params=pltpu.CompilerParams(use_tc_tiling_on_sc=False),
    )
    out = run(embeddings, x.reshape(ROWS, L))
    return out.reshape(BATCH, HIST, D)


# speedup vs baseline: 1.7641x; 1.6033x over previous
import functools

import jax
import jax.numpy as jnp
from jax import lax
from jax.experimental import pallas as pl
from jax.experimental.pallas import tpu as pltpu, tpu_sc as plsc

VOCAB = 1_000_000
D = 32
NC, NS = 2, 16
NW = NC * NS

BATCH, HIST = 16384, 50
ROWS_PER_W = BATCH // NW
CB = 16
UNITS = ROWS_PER_W // CB


def _gather_body(table_hbm, idx_hbm, out_hbm, idx_v, rows_v, gsem, osem):
    wid = lax.axis_index("s") * NC + lax.axis_index("c")
    w_row0 = wid * ROWS_PER_W

    def fire(u, buf):
        r0 = pl.multiple_of(w_row0 + u * CB, 8)
        ib = pl.multiple_of(buf * CB, 8)
        pltpu.sync_copy(idx_hbm.at[pl.ds(r0, CB)], idx_v.at[pl.ds(ib, CB)])
        for c in range(CB):
            pltpu.async_copy(
                table_hbm.at[idx_v.at[ib + c]],
                rows_v.at[ib + c],
                gsem,
            )

    def wait_gathers(buf):
        pltpu.make_async_copy(
            out_hbm.at[pl.ds(0, CB)],
            rows_v.at[pl.ds(pl.multiple_of(buf * CB, 8), CB)],
            gsem,
        ).wait()

    def out_copy(u, buf):
        r0 = pl.multiple_of(w_row0 + u * CB, 8)
        return pltpu.make_async_copy(
            rows_v.at[pl.ds(pl.multiple_of(buf * CB, 8), CB)],
            out_hbm.at[pl.ds(r0, CB)],
            osem,
        )

    fire(0, 0)

    def body(u, _):
        buf = lax.rem(u, 2)
        wait_gathers(buf)

        @pl.when(u >= 1)
        def _():
            out_copy(u - 1, 1 - buf).wait()

        @pl.when(u + 1 < UNITS)
        def _():
            fire(u + 1, 1 - buf)

        out_copy(u, buf).start()
        return ()

    lax.fori_loop(0, UNITS, body, (), unroll=False)
    out_copy(UNITS - 1, lax.rem(UNITS - 1, 2)).wait()


@functools.partial(jax.jit, static_argnames=())
def kernel(x, embeddings):
    mesh = plsc.VectorSubcoreMesh(core_axis_name="c", subcore_axis_name="s")
    run = pl.kernel(
        _gather_body,
        out_type=jax.ShapeDtypeStruct((BATCH, HIST, D), jnp.float32),
        mesh=mesh,
        scratch_types=[
            pltpu.VMEM((2 * CB, HIST), jnp.int32),
            pltpu.VMEM((2 * CB, HIST, D), jnp.float32),
            pltpu.SemaphoreType.DMA,
            pltpu.SemaphoreType.DMA,
        ],
        compiler_params=pltpu.CompilerParams(use_tc_tiling_on_sc=False),
    )
    return run(embeddings, x)
